# baseline (device time: 101218 ns/iter reference)
import jax
import jax.numpy as jnp
from jax import lax
from jax.experimental import pallas as pl
from jax.experimental.pallas import tpu as pltpu

N_DEV = 4

M = 64
KX = 2048
H = 4096
NB = 512
N_IN = H // NB
HALF = H // 2
KT = HALF
NT = KX // 4
N_NT = KX // NT


def kernel(x, Win0, Wout0, Win1, Wout1, Win2, Wout2):
    bf16 = jnp.bfloat16
    f32 = jnp.float32

    def body(x_ref, win0, wout0, win1, wout1, win2, wout2, o_ref,
             xa, hb, pair, buf1, buf2, xacc, win_buf, wout_buf,
             win_sem, wout_sem, s1, r1, s2, r2):
        me = lax.axis_index("i")
        p1 = me ^ 1
        p2 = me ^ 2
        wins = [win0, win1, win2]
        wouts = [wout0, wout1, wout2]

        def win_dma(l, j, slot):
            return pltpu.make_async_copy(
                wins[l].at[:, pl.ds(j * NB, NB)],
                win_buf.at[slot],
                win_sem.at[slot],
            )

        def wout_dma(l, ka, nb):
            t = ka * N_NT + nb
            return pltpu.make_async_copy(
                wouts[l].at[pl.ds(ka * KT, KT), pl.ds(nb * NT, NT)],
                wout_buf.at[t],
                wout_sem.at[t],
            )

        def rdma1(c):
            return pltpu.make_async_remote_copy(
                src_ref=hb.at[:, pl.ds(c * HALF, HALF)],
                dst_ref=buf1.at[:, pl.ds(c * HALF, HALF)],
                send_sem=s1.at[c], recv_sem=r1.at[c],
                device_id=(p1,), device_id_type=pl.DeviceIdType.MESH,
            )

        def rdma2(c):
            return pltpu.make_async_remote_copy(
                src_ref=pair.at[:, pl.ds(c * HALF, HALF)],
                dst_ref=buf2.at[:, pl.ds(c * HALF, HALF)],
                send_sem=s2.at[c], recv_sem=r2.at[c],
                device_id=(p2,), device_id_type=pl.DeviceIdType.MESH,
            )

        def half(ref, c):
            return ref[:, pl.ds(c * HALF, HALF)]

        for j in range(6):
            win_dma(0, j, j).start()
        for ka in range(2):
            for nb in range(N_NT):
                wout_dma(0, ka, nb).start()
        xa[...] = x_ref[...].astype(bf16)

        for l in range(3):
            for j in range(N_IN):
                win_dma(l, j, j % 6).wait()
                wb = win_buf[j % 6].astype(bf16)
                hb[:, pl.ds(j * NB, NB)] = jnp.dot(
                    xa[...], wb, preferred_element_type=f32
                ).astype(bf16)
                if j + 6 < N_IN:
                    win_dma(l, j + 6, j % 6).start()
                if j == N_IN // 2 - 1:
                    rdma1(0).start()
            rdma1(1).start()
            if l < 2:
                for j in range(6):
                    win_dma(l + 1, j, j).start()

            r1a = rdma1(0)
            r1a.wait()
            pair[:, pl.ds(0, HALF)] = half(hb, 0) + half(buf1, 0)
            rdma2(0).start()
            r1b = rdma1(1)
            r1b.wait()
            pair[:, pl.ds(HALF, HALF)] = half(hb, 1) + half(buf1, 1)
            rdma2(1).start()

            rdma2(0).wait()
            hb[:, pl.ds(0, HALF)] = jnp.maximum(
                half(pair, 0) + half(buf2, 0), jnp.bfloat16(0.0))
            for nb in range(N_NT):
                wout_dma(l, 0, nb).wait()
                wb = wout_buf[nb].astype(bf16)
                xacc[:, pl.ds(nb * NT, NT)] = jnp.dot(
                    half(hb, 0), wb, preferred_element_type=f32
                )

            rdma2(1).wait()
            hb[:, pl.ds(HALF, HALF)] = jnp.maximum(
                half(pair, 1) + half(buf2, 1), jnp.bfloat16(0.0))
            for nb in range(N_NT):
                wout_dma(l, 1, nb).wait()
                wb = wout_buf[N_NT + nb].astype(bf16)
                xacc[:, pl.ds(nb * NT, NT)] = xacc[:, pl.ds(nb * NT, NT)] + jnp.dot(
                    half(hb, 1), wb, preferred_element_type=f32
                )
                if l < 2:
                    wout_dma(l + 1, 0, nb).start()
                    wout_dma(l + 1, 1, nb).start()

            if l == 2:
                o_ref[...] = xacc[...]
            else:
                xa[...] = xacc[...].astype(bf16)

    in_specs = [
        pl.BlockSpec(memory_space=pltpu.VMEM),
        pl.BlockSpec(memory_space=pltpu.MemorySpace.HBM),
        pl.BlockSpec(memory_space=pltpu.MemorySpace.HBM),
        pl.BlockSpec(memory_space=pltpu.MemorySpace.HBM),
        pl.BlockSpec(memory_space=pltpu.MemorySpace.HBM),
        pl.BlockSpec(memory_space=pltpu.MemorySpace.HBM),
        pl.BlockSpec(memory_space=pltpu.MemorySpace.HBM),
    ]
    scratch_shapes = [
        pltpu.VMEM((M, KX), jnp.bfloat16),
        pltpu.VMEM((M, H), jnp.bfloat16),
        pltpu.VMEM((M, H), jnp.bfloat16),
        pltpu.VMEM((M, H), jnp.bfloat16),
        pltpu.VMEM((M, H), jnp.bfloat16),
        pltpu.VMEM((M, KX), jnp.float32),
        pltpu.VMEM((6, KX, NB), jnp.float32),
        pltpu.VMEM((2 * N_NT, KT, NT), jnp.float32),
        pltpu.SemaphoreType.DMA((6,)),
        pltpu.SemaphoreType.DMA((2 * N_NT,)),
        pltpu.SemaphoreType.DMA((2,)),
        pltpu.SemaphoreType.DMA((2,)),
        pltpu.SemaphoreType.DMA((2,)),
        pltpu.SemaphoreType.DMA((2,)),
    ]

    return pl.pallas_call(
        body,
        in_specs=in_specs,
        out_specs=pl.BlockSpec(memory_space=pltpu.VMEM),
        scratch_shapes=scratch_shapes,
        compiler_params=pltpu.CompilerParams(vmem_limit_bytes=100 * 1024 * 1024),
        out_shape=jax.ShapeDtypeStruct((M, KX), jnp.float32),
    )(x, Win0, Wout0, Win1, Wout1, Win2, Wout2)


# device time: 86863 ns/iter; 1.1653x vs baseline; 1.1653x over previous
import jax
import jax.numpy as jnp
from jax import lax
from jax.experimental import pallas as pl
from jax.experimental.pallas import tpu as pltpu

N_DEV = 4

M = 64
KX = 2048
H = 4096
NB = 512
N_IN = H // NB
HALF = H // 2
KT = HALF
NT = KX // 4
N_NT = KX // NT


def kernel(x, Win0, Wout0, Win1, Wout1, Win2, Wout2):
    bf16 = jnp.bfloat16
    f32 = jnp.float32

    def body(x_ref, win0, wout0, win1, wout1, win2, wout2, o_ref,
             xa, hb, pair, buf1, buf2, xacc, win_buf, wout_buf,
             win_sem, wout_sem, s1, r1, s2, r2):
        me = lax.axis_index("i")
        p1 = me ^ 1
        p2 = me ^ 2
        wins = [win0, win1, win2]
        wouts = [wout0, wout1, wout2]

        def win_dma(l, j, slot):
            return pltpu.make_async_copy(
                wins[l].at[:, pl.ds(j * NB, NB)],
                win_buf.at[slot],
                win_sem.at[slot],
            )

        def wout_dma(l, ka, nb):
            t = ka * N_NT + nb
            return pltpu.make_async_copy(
                wouts[l].at[pl.ds(ka * KT, KT), pl.ds(nb * NT, NT)],
                wout_buf.at[t],
                wout_sem.at[t],
            )

        def rdma1(c):
            return pltpu.make_async_remote_copy(
                src_ref=hb.at[:, pl.ds(c * HALF, HALF)],
                dst_ref=buf1.at[:, pl.ds(c * HALF, HALF)],
                send_sem=s1.at[c], recv_sem=r1.at[c],
                device_id=(p1,), device_id_type=pl.DeviceIdType.MESH,
            )

        def rdma2(c):
            return pltpu.make_async_remote_copy(
                src_ref=pair.at[:, pl.ds(c * HALF, HALF)],
                dst_ref=buf2.at[:, pl.ds(c * HALF, HALF)],
                send_sem=s2.at[c], recv_sem=r2.at[c],
                device_id=(p2,), device_id_type=pl.DeviceIdType.MESH,
            )

        def half(ref, c):
            return ref[:, pl.ds(c * HALF, HALF)]

        for j in range(6):
            win_dma(0, j, j).start()
        xa[...] = x_ref[...].astype(bf16)

        for l in range(3):
            for j in range(N_IN):
                win_dma(l, j, j % 6).wait()
                wb = win_buf[j % 6].astype(bf16)
                hb[:, pl.ds(j * NB, NB)] = jnp.dot(
                    xa[...], wb, preferred_element_type=f32
                ).astype(bf16)
                if j + 6 < N_IN:
                    win_dma(l, j + 6, j % 6).start()
                if j == 1:
                    for nb in range(N_NT):
                        wout_dma(l, 0, nb).start()
                elif j == 3:
                    for nb in range(N_NT):
                        wout_dma(l, 1, nb).start()
                if j == N_IN // 2 - 1:
                    rdma1(0).start()
            rdma1(1).start()
            if l < 2:
                for j in range(6):
                    win_dma(l + 1, j, j).start()

            r1a = rdma1(0)
            r1a.wait()
            pair[:, pl.ds(0, HALF)] = half(hb, 0) + half(buf1, 0)
            rdma2(0).start()
            r1b = rdma1(1)
            r1b.wait()
            pair[:, pl.ds(HALF, HALF)] = half(hb, 1) + half(buf1, 1)
            rdma2(1).start()

            rdma2(0).wait()
            hb[:, pl.ds(0, HALF)] = jnp.maximum(
                half(pair, 0) + half(buf2, 0), jnp.bfloat16(0.0))
            for nb in range(N_NT):
                wout_dma(l, 0, nb).wait()
                wb = wout_buf[nb].astype(bf16)
                xacc[:, pl.ds(nb * NT, NT)] = jnp.dot(
                    half(hb, 0), wb, preferred_element_type=f32
                )

            rdma2(1).wait()
            hb[:, pl.ds(HALF, HALF)] = jnp.maximum(
                half(pair, 1) + half(buf2, 1), jnp.bfloat16(0.0))
            for nb in range(N_NT):
                wout_dma(l, 1, nb).wait()
                wb = wout_buf[N_NT + nb].astype(bf16)
                xacc[:, pl.ds(nb * NT, NT)] = xacc[:, pl.ds(nb * NT, NT)] + jnp.dot(
                    half(hb, 1), wb, preferred_element_type=f32
                )

            if l == 2:
                o_ref[...] = xacc[...]
            else:
                xa[...] = xacc[...].astype(bf16)

    in_specs = [
        pl.BlockSpec(memory_space=pltpu.VMEM),
        pl.BlockSpec(memory_space=pltpu.MemorySpace.HBM),
        pl.BlockSpec(memory_space=pltpu.MemorySpace.HBM),
        pl.BlockSpec(memory_space=pltpu.MemorySpace.HBM),
        pl.BlockSpec(memory_space=pltpu.MemorySpace.HBM),
        pl.BlockSpec(memory_space=pltpu.MemorySpace.HBM),
        pl.BlockSpec(memory_space=pltpu.MemorySpace.HBM),
    ]
    scratch_shapes = [
        pltpu.VMEM((M, KX), jnp.bfloat16),
        pltpu.VMEM((M, H), jnp.bfloat16),
        pltpu.VMEM((M, H), jnp.bfloat16),
        pltpu.VMEM((M, H), jnp.bfloat16),
        pltpu.VMEM((M, H), jnp.bfloat16),
        pltpu.VMEM((M, KX), jnp.float32),
        pltpu.VMEM((6, KX, NB), jnp.float32),
        pltpu.VMEM((2 * N_NT, KT, NT), jnp.float32),
        pltpu.SemaphoreType.DMA((6,)),
        pltpu.SemaphoreType.DMA((2 * N_NT,)),
        pltpu.SemaphoreType.DMA((2,)),
        pltpu.SemaphoreType.DMA((2,)),
        pltpu.SemaphoreType.DMA((2,)),
        pltpu.SemaphoreType.DMA((2,)),
    ]

    return pl.pallas_call(
        body,
        in_specs=in_specs,
        out_specs=pl.BlockSpec(memory_space=pltpu.VMEM),
        scratch_shapes=scratch_shapes,
        compiler_params=pltpu.CompilerParams(vmem_limit_bytes=100 * 1024 * 1024),
        out_shape=jax.ShapeDtypeStruct((M, KX), jnp.float32),
    )(x, Win0, Wout0, Win1, Wout1, Win2, Wout2)


# device time: 77596 ns/iter; 1.3044x vs baseline; 1.1194x over previous
import jax
import jax.numpy as jnp
from jax import lax
from jax.experimental import pallas as pl
from jax.experimental.pallas import tpu as pltpu

N_DEV = 4

M = 64
KX = 2048
H = 4096
NB = 512
N_IN = H // NB
HALF = H // 2
KT = HALF
NT = KX // 2


def kernel(x, Win0, Wout0, Win1, Wout1, Win2, Wout2):
    bf16 = jnp.bfloat16
    f32 = jnp.float32

    def body(x_ref, win0, wout0, win1, wout1, win2, wout2, o_ref,
             xa, hb, pair, buf1, buf2, xacc, win_buf, wout_buf,
             win_sem, wout_sem, s1, r1, s2, r2):
        me = lax.axis_index("i")
        p1 = me ^ 1
        p2 = me ^ 2
        wins = [win0, win1, win2]
        wouts = [wout0, wout1, wout2]

        def win_dma(l, j, slot):
            return pltpu.make_async_copy(
                wins[l].at[:, pl.ds(j * NB, NB)],
                win_buf.at[slot],
                win_sem.at[slot],
            )

        def wout_dma(l, ka, nb):
            t = ka * 2 + nb
            return pltpu.make_async_copy(
                wouts[l].at[pl.ds(ka * KT, KT), pl.ds(nb * NT, NT)],
                wout_buf.at[t],
                wout_sem.at[t],
            )

        def rdma1(c):
            return pltpu.make_async_remote_copy(
                src_ref=hb.at[:, pl.ds(c * HALF, HALF)],
                dst_ref=buf1.at[:, pl.ds(c * HALF, HALF)],
                send_sem=s1.at[c], recv_sem=r1.at[c],
                device_id=(p1,), device_id_type=pl.DeviceIdType.MESH,
            )

        def rdma2(c):
            return pltpu.make_async_remote_copy(
                src_ref=pair.at[:, pl.ds(c * HALF, HALF)],
                dst_ref=buf2.at[:, pl.ds(c * HALF, HALF)],
                send_sem=s2.at[c], recv_sem=r2.at[c],
                device_id=(p2,), device_id_type=pl.DeviceIdType.MESH,
            )

        def half(ref, c):
            return ref[:, pl.ds(c * HALF, HALF)]

        for j in range(5):
            win_dma(0, j, j).start()
        xa[...] = x_ref[...].astype(bf16)

        for l in range(3):
            for j in range(N_IN):
                slot = j % 6
                nxt = j + 5
                if nxt < N_IN:
                    win_dma(l, nxt, nxt % 6).start()
                elif j == N_IN - 1:
                    for ka in range(2):
                        for nb in range(2):
                            wout_dma(l, ka, nb).start()
                    if l < 2:
                        win_dma(l + 1, 0, 0).start()
                win_dma(l, j, slot).wait()
                wb = win_buf[slot].astype(bf16)
                hb[:, pl.ds(j * NB, NB)] = jnp.dot(
                    xa[...], wb, preferred_element_type=f32
                ).astype(bf16)
                if j == N_IN // 2 - 1:
                    rdma1(0).start()
            rdma1(1).start()
            if l < 2:
                for k in range(1, 5):
                    win_dma(l + 1, k, k).start()

            r1a = rdma1(0)
            r1a.wait()
            pair[:, pl.ds(0, HALF)] = half(hb, 0) + half(buf1, 0)
            rdma2(0).start()
            r1b = rdma1(1)
            r1b.wait()
            pair[:, pl.ds(HALF, HALF)] = half(hb, 1) + half(buf1, 1)
            rdma2(1).start()

            rdma2(0).wait()
            hb[:, pl.ds(0, HALF)] = jnp.maximum(
                half(pair, 0) + half(buf2, 0), jnp.bfloat16(0.0))
            for nb in range(2):
                wout_dma(l, 0, nb).wait()
                wb = wout_buf[nb].astype(bf16)
                xacc[:, pl.ds(nb * NT, NT)] = jnp.dot(
                    half(hb, 0), wb, preferred_element_type=f32
                )

            rdma2(1).wait()
            hb[:, pl.ds(HALF, HALF)] = jnp.maximum(
                half(pair, 1) + half(buf2, 1), jnp.bfloat16(0.0))
            for nb in range(2):
                wout_dma(l, 1, nb).wait()
                wb = wout_buf[2 + nb].astype(bf16)
                xacc[:, pl.ds(nb * NT, NT)] = xacc[:, pl.ds(nb * NT, NT)] + jnp.dot(
                    half(hb, 1), wb, preferred_element_type=f32
                )

            if l == 2:
                o_ref[...] = xacc[...]
            else:
                xa[...] = xacc[...].astype(bf16)

    in_specs = [
        pl.BlockSpec(memory_space=pltpu.VMEM),
        pl.BlockSpec(memory_space=pltpu.MemorySpace.HBM),
        pl.BlockSpec(memory_space=pltpu.MemorySpace.HBM),
        pl.BlockSpec(memory_space=pltpu.MemorySpace.HBM),
        pl.BlockSpec(memory_space=pltpu.MemorySpace.HBM),
        pl.BlockSpec(memory_space=pltpu.MemorySpace.HBM),
        pl.BlockSpec(memory_space=pltpu.MemorySpace.HBM),
    ]
    scratch_shapes = [
        pltpu.VMEM((M, KX), jnp.bfloat16),
        pltpu.VMEM((M, H), jnp.bfloat16),
        pltpu.VMEM((M, H), jnp.bfloat16),
        pltpu.VMEM((M, H), jnp.bfloat16),
        pltpu.VMEM((M, H), jnp.bfloat16),
        pltpu.VMEM((M, KX), jnp.float32),
        pltpu.VMEM((6, KX, NB), jnp.float32),
        pltpu.VMEM((4, KT, NT), jnp.float32),
        pltpu.SemaphoreType.DMA((6,)),
        pltpu.SemaphoreType.DMA((4,)),
        pltpu.SemaphoreType.DMA((2,)),
        pltpu.SemaphoreType.DMA((2,)),
        pltpu.SemaphoreType.DMA((2,)),
        pltpu.SemaphoreType.DMA((2,)),
    ]

    return pl.pallas_call(
        body,
        in_specs=in_specs,
        out_specs=pl.BlockSpec(memory_space=pltpu.VMEM),
        scratch_shapes=scratch_shapes,
        compiler_params=pltpu.CompilerParams(vmem_limit_bytes=100 * 1024 * 1024),
        out_shape=jax.ShapeDtypeStruct((M, KX), jnp.float32),
    )(x, Win0, Wout0, Win1, Wout1, Win2, Wout2)
